# baseline (speedup 1.0000x reference)
import functools

import jax
import jax.numpy as jnp
from jax.experimental import pallas as pl
from jax.experimental.pallas import tpu as pltpu

_N = 200000
_R = 3125


def _loss_body(a_ref, y_ref, out_ref):
    a = a_ref[...]
    yf = y_ref[...].astype(jnp.float32)

    lane = jax.lax.broadcasted_iota(jnp.int32, (64, 128), 1)
    sub = jax.lax.broadcasted_iota(jnp.int32, (64, 128), 0)
    expand = (lane // 2 == sub).astype(jnp.float32)
    yx = jax.lax.dot_general(yf, expand, (((1,), (0,)), ((), ())),
                             preferred_element_type=jnp.float32)

    ar = pltpu.roll(a, 127, 1)
    m = jnp.maximum(a, ar)
    sp = jnp.log1p(jnp.exp(-jnp.abs(a - ar)))
    chosen = jnp.where(yx == 1.0, ar, a)
    loss = jnp.maximum(m - chosen + sp, 0.0)

    even = jax.lax.broadcasted_iota(jnp.int32, (_R, 128), 1) % 2 == 0
    zeros = jnp.zeros_like(loss)
    pos = even & (yx == 1.0)
    neg = even & (yx == 0.0)
    n_pos = jnp.sum(yf)
    n_neg = jnp.float32(_N) - n_pos
    pos_sum = jnp.sum(jnp.where(pos, loss, zeros))
    neg_sum = jnp.sum(jnp.where(neg, loss, zeros))

    out_ref[0] = pos_sum / n_pos + neg_sum / n_neg

    @pl.when(n_neg > 3.0 * n_pos)
    def _rare():
        k = 3.0 * n_pos
        negloss = jnp.where(neg, loss, -1.0)
        bits = jax.lax.bitcast_convert_type(negloss, jnp.int32)
        ki = k.astype(jnp.int32)
        onesf = jnp.ones_like(loss)

        def step(_, lohi):
            lo, hi = lohi
            mid = (lo + hi) // 2
            cnt = jnp.sum(jnp.where(bits >= mid, onesf, zeros)).astype(
                jnp.int32)
            take = cnt >= ki
            return jnp.where(take, mid, lo), jnp.where(take, hi, mid)

        lo0 = jnp.int32(0)
        hi0 = jnp.int32(0x7F800000)
        lo, _ = jax.lax.fori_loop(0, 31, step, (lo0, hi0))
        thr = jax.lax.bitcast_convert_type(lo, jnp.float32)
        gt = bits > lo
        cnt_gt = jnp.sum(jnp.where(gt, onesf, zeros))
        sum_gt = jnp.sum(jnp.where(gt, negloss, zeros))
        topk_sum = sum_gt + (k - cnt_gt) * thr
        out_ref[0] = pos_sum / n_pos + topk_sum / k


@jax.jit
def _rpn_cls_loss(a, y):
    out = pl.pallas_call(
        _loss_body,
        out_shape=jax.ShapeDtypeStruct((1,), jnp.float32),
        in_specs=[
            pl.BlockSpec(memory_space=pltpu.VMEM),
            pl.BlockSpec(memory_space=pltpu.VMEM),
        ],
        out_specs=pl.BlockSpec(memory_space=pltpu.SMEM),
    )(a, y)
    return out[0]


def kernel(cls, regr, refi, target_cls, target_regr, target_refi):
    a = jnp.reshape(cls, (_R, 128))
    y = jnp.reshape(target_cls, (_R, 64))
    return _rpn_cls_loss(a, y)
